# Initial kernel scaffold; baseline (speedup 1.0000x reference)
import jax
import jax.numpy as jnp
from jax.experimental import pallas as pl


def kernel(x, edge_index, edge_weight, edge_attr, lin1_w, fn1_w, fn1_b, fn2_w, fn2_b, lin2_w, lin2_b, lin_w, lin_b):
    raise NotImplementedError("write your pallas kernel here")


# speedup vs baseline: 1.8853x; 1.8853x over previous
import functools
import math

import jax
import jax.numpy as jnp
from jax import lax
from jax.experimental import pallas as pl
from jax.experimental.pallas import tpu as pltpu
from jax.experimental.pallas import tpu_sc as plsc

N = 10000
E = 320000
HC = 128
NRBF = 16
CUTOFF = 10.0

NC = 2
NS = 16
L = 16
NW = NC * NS

CHUNK = 40
EDGES_PER_W = E // NW
NCHUNK = EDGES_PER_W // CHUNK

RBLK = 8
NRBLK = N // RBLK

_TE = 8000


def _h_body(x_ref, w_ref, o_ref):
    o_ref[...] = jnp.dot(x_ref[...], w_ref[...],
                         preferred_element_type=jnp.float32)


def _compute_h(x, lin1_w):
    return pl.pallas_call(
        _h_body,
        out_shape=jax.ShapeDtypeStruct((N, HC), jnp.float32),
    )(x, lin1_w)


def _filter_body(ea_ref, ew_ref, w1_ref, b1_ref, w2_ref, b2_ref, o_ref):
    t = jnp.tanh(jnp.dot(ea_ref[...], w1_ref[...],
                         preferred_element_type=jnp.float32) + b1_ref[...])
    w = jnp.dot(t, w2_ref[...], preferred_element_type=jnp.float32) + b2_ref[...]
    ew = ew_ref[...]
    c = 0.5 * (jnp.cos(ew * (math.pi / CUTOFF)) + 1.0)
    c = c * (ew < CUTOFF).astype(jnp.float32)
    o_ref[...] = w * c


def _compute_w(edge_attr, edge_weight, fn1_w, fn1_b, fn2_w, fn2_b):
    grid = E // _TE
    return pl.pallas_call(
        _filter_body,
        grid=(grid,),
        in_specs=[
            pl.BlockSpec((_TE, NRBF), lambda i: (i, 0)),
            pl.BlockSpec((_TE, 1), lambda i: (i, 0)),
            pl.BlockSpec((NRBF, HC), lambda i: (0, 0)),
            pl.BlockSpec((1, HC), lambda i: (0, 0)),
            pl.BlockSpec((HC, HC), lambda i: (0, 0)),
            pl.BlockSpec((1, HC), lambda i: (0, 0)),
        ],
        out_specs=pl.BlockSpec((_TE, HC), lambda i: (i, 0)),
        out_shape=jax.ShapeDtypeStruct((E, HC), jnp.float32),
    )(edge_attr, edge_weight.reshape(E, 1), fn1_w, fn1_b.reshape(1, HC),
      fn2_w, fn2_b.reshape(1, HC))


def _sc_conv_body(h_hbm, w_hbm, src_hbm, dst_hbm, out_hbm,
                  src0, src1, dstS0, dstS1, rows0, rows1, wv0, wv1, acc_sh,
                  sem_i0, sem_i1, sem_d0, sem_d1, sem_g0, sem_g1,
                  sem_w0, sem_w1, sem_s0, sem_s1):
    cid = lax.axis_index("c")
    sid = lax.axis_index("s")
    wid = cid * NS + sid
    base = wid * EDGES_PER_W

    srcb = (src0, src1)
    dstb = (dstS0, dstS1)
    rowsb = (rows0, rows1)
    wb = (wv0, wv1)
    sem_i = (sem_i0, sem_i1)
    sem_d = (sem_d0, sem_d1)
    sem_g = (sem_g0, sem_g1)
    sem_w = (sem_w0, sem_w1)
    sem_s = (sem_s0, sem_s1)

    @pl.loop(0, RBLK)
    def _zero_rows(r):
        @pl.loop(0, HC, step=L)
        def _zero_lanes(k):
            rows0[r, pl.ds(k, L)] = jnp.zeros((L,), jnp.float32)

    @pl.loop(sid, NRBLK, step=NS)
    def _zero_acc(b):
        pltpu.sync_copy(rows0.at[pl.ds(0, RBLK)],
                        acc_sh.at[pl.ds(b * RBLK, RBLK)])

    plsc.subcore_barrier()

    def issue_src(c, p):
        pltpu.async_copy(src_hbm.at[pl.ds(base + c * CHUNK, CHUNK)],
                         srcb[p], sem_i[p])

    def wait_src(p):
        pltpu.make_async_copy(src_hbm.at[pl.ds(0, CHUNK)], srcb[p],
                              sem_i[p]).wait()

    def issue_dst(c, p):
        pltpu.async_copy(dst_hbm.at[pl.ds(base + c * CHUNK, CHUNK)],
                         dstb[p], sem_d[p])

    def wait_dst(p):
        pltpu.make_async_copy(dst_hbm.at[pl.ds(0, CHUNK)], dstb[p],
                              sem_d[p]).wait()

    def issue_gw(c, p):
        pltpu.async_copy(h_hbm.at[srcb[p]], rowsb[p], sem_g[p])
        pltpu.async_copy(w_hbm.at[pl.ds(base + c * CHUNK, CHUNK)],
                         wb[p], sem_w[p])

    def wait_gw(p):
        pltpu.make_async_copy(h_hbm.at[srcb[p]], rowsb[p], sem_g[p]).wait()
        pltpu.make_async_copy(w_hbm.at[pl.ds(0, CHUNK)], wb[p],
                              sem_w[p]).wait()

    def issue_scatter(p):
        pltpu.async_copy(rowsb[p], acc_sh.at[dstb[p]], sem_s[p], add=True)

    def wait_scatter(p):
        pltpu.make_async_copy(rowsb[p], acc_sh.at[dstb[p]], sem_s[p]).wait()

    def multiply(p):
        rv, wv = rowsb[p], wb[p]

        @pl.loop(0, CHUNK)
        def _row(r):
            @pl.loop(0, HC, step=L)
            def _lane(k):
                rv[r, pl.ds(k, L)] = rv[r, pl.ds(k, L)] * wv[r, pl.ds(k, L)]

    def stage(c, p, first=False, issue_next=True, issue_src2=True):
        q = 1 - p
        if not first:
            wait_scatter(q)
        if issue_next:
            wait_src(q)
            issue_gw(c + 1, q)
            issue_dst(c + 1, q)
        wait_gw(p)
        if issue_src2:
            issue_src(c + 2, p)
        multiply(p)
        wait_dst(p)
        issue_scatter(p)

    issue_src(0, 0)
    issue_src(1, 1)
    issue_dst(0, 0)
    wait_src(0)
    issue_gw(0, 0)

    stage(0, 0, first=True)
    stage(1, 1)

    @pl.loop(1, NCHUNK // 2 - 1)
    def _pair(t):
        c0 = 2 * t
        stage(c0, 0)
        stage(c0 + 1, 1)

    stage(NCHUNK - 2, 0, issue_src2=False)
    stage(NCHUNK - 1, 1, issue_next=False, issue_src2=False)
    wait_scatter(1)

    plsc.subcore_barrier()

    @pl.loop(sid, NRBLK, step=NS)
    def _writeback(b):
        pltpu.sync_copy(acc_sh.at[pl.ds(b * RBLK, RBLK)],
                        out_hbm.at[pl.ds(cid * N + b * RBLK, RBLK)])


def _sc_conv(h, wmat, src, dst):
    mesh = plsc.VectorSubcoreMesh(core_axis_name="c", subcore_axis_name="s",
                                  num_cores=NC, num_subcores=NS)
    kern = pl.kernel(
        _sc_conv_body,
        out_type=jax.ShapeDtypeStruct((NC * N, HC), jnp.float32),
        mesh=mesh,
        scratch_types=[
            pltpu.VMEM((CHUNK,), jnp.int32),
            pltpu.VMEM((CHUNK,), jnp.int32),
            pltpu.VMEM((CHUNK,), jnp.int32),
            pltpu.VMEM((CHUNK,), jnp.int32),
            pltpu.VMEM((CHUNK, HC), jnp.float32),
            pltpu.VMEM((CHUNK, HC), jnp.float32),
            pltpu.VMEM((CHUNK, HC), jnp.float32),
            pltpu.VMEM((CHUNK, HC), jnp.float32),
            pltpu.VMEM_SHARED((N, HC), jnp.float32),
        ] + [pltpu.SemaphoreType.DMA] * 10,
    )
    return kern(h, wmat, src, dst)


def _out_body(p_ref, w2_ref, b2_ref, wo_ref, bo_ref, o_ref):
    agg = p_ref[:N, :] + p_ref[N:, :]
    conv = jnp.dot(agg, w2_ref[...], preferred_element_type=jnp.float32) + b2_ref[...]
    o_ref[...] = jnp.dot(jnp.tanh(conv), wo_ref[...],
                         preferred_element_type=jnp.float32) + bo_ref[...]


def _compute_out(parts, lin2_w, lin2_b, lin_w, lin_b):
    return pl.pallas_call(
        _out_body,
        out_shape=jax.ShapeDtypeStruct((N, HC), jnp.float32),
    )(parts, lin2_w, lin2_b.reshape(1, HC), lin_w, lin_b.reshape(1, HC))


def kernel(x, edge_index, edge_weight, edge_attr,
           lin1_w, fn1_w, fn1_b, fn2_w, fn2_b, lin2_w, lin2_b, lin_w, lin_b):
    src = edge_index[0]
    dst = edge_index[1]
    h = _compute_h(x, lin1_w)
    wmat = _compute_w(edge_attr, edge_weight, fn1_w, fn1_b, fn2_w, fn2_b)
    parts = _sc_conv(h, wmat, src, dst)
    return _compute_out(parts, lin2_w, lin2_b, lin_w, lin_b)
